# baseline (speedup 1.0000x reference)
import functools

import jax
import jax.numpy as jnp
from jax import lax
from jax.experimental import pallas as pl
from jax.experimental.pallas import tpu as pltpu
from jax.experimental.pallas import tpu_sc as plsc

NEGK = 50
TEMP = 0.05
NEG_FILL = -3.0e38
IMIN = -2147483648
IMAX = 2147483647


def _to_key(s):
    bits = lax.bitcast_convert_type(s, jnp.int32)
    return jnp.where(bits < 0, bits ^ IMAX, bits)


def _from_key(k):
    bits = jnp.where(k < 0, k ^ IMAX, k)
    return lax.bitcast_convert_type(bits, jnp.float32)


def _mid(lo, hi):
    return (lo >> 1) + (hi >> 1) + (lo & hi & 1) + ((lo ^ hi) & 1)


def _loss_block(feat_ref, pm_ref, labels_ref, blab_ref, out_ref, keys_ref):
    rb, d = feat_ref.shape
    m = pm_ref.shape[0]
    cb = min(512, m)
    n_chunks = m // cb

    f = feat_ref[...] * (1.0 / TEMP)
    blab = blab_ref[...]

    def chunk(c, carry):
        n_pos, sum_pos, pos_max, pos_se, g = carry
        pm_c = pm_ref[pl.ds(c * cb, cb), :]
        s = lax.dot_general(f, pm_c, (((1,), (1,)), ((), ())),
                            preferred_element_type=jnp.float32)
        mask = labels_ref[:, pl.ds(c * cb, cb)] == blab
        key = jnp.where(mask, IMIN, _to_key(s))
        keys_ref[:, pl.ds(c * cb, cb)] = key
        g = jnp.maximum(g, key)
        n_pos = n_pos + jnp.sum(mask.astype(jnp.int32), axis=1, keepdims=True)
        sum_pos = sum_pos + jnp.sum(jnp.where(mask, s, 0.0), axis=1, keepdims=True)
        m_new = jnp.maximum(
            pos_max, jnp.max(jnp.where(mask, s, NEG_FILL), axis=1, keepdims=True))
        pos_se = pos_se * jnp.exp(pos_max - m_new) + jnp.sum(
            jnp.where(mask, jnp.exp(s - m_new), 0.0), axis=1, keepdims=True)
        return n_pos, sum_pos, m_new, pos_se, g

    zero = jnp.zeros((rb, 1), jnp.float32)
    n_pos, sum_pos, pos_max, pos_se, g = lax.fori_loop(
        0, n_chunks, chunk,
        (jnp.zeros((rb, 1), jnp.int32), zero, jnp.full((rb, 1), NEG_FILL), zero,
         jnp.full((rb, cb), IMIN, jnp.int32)))

    hi0 = jnp.max(g, axis=1, keepdims=True)
    kneg = jnp.maximum(NEGK - jnp.minimum(n_pos, NEGK), 0)

    def gsearch(_, carry):
        lo, hi = carry
        mid = _mid(lo, hi)
        cnt = jnp.sum((g >= mid).astype(jnp.int32), axis=1, keepdims=True)
        ge = cnt >= (NEGK + 1)
        return jnp.where(ge, mid, lo), jnp.where(ge, hi, mid - 1)

    g51, _ = lax.fori_loop(
        0, 32, gsearch, (jnp.min(g, axis=1, keepdims=True), hi0))

    def scond(carry):
        lo, hi = carry
        return jnp.any(lo < hi)

    def sstep(lo, hi):
        mid = _mid(lo, hi)
        cnt = jnp.sum((keys_ref[...] >= mid).astype(jnp.int32),
                      axis=1, keepdims=True)
        ge = cnt >= kneg
        return jnp.where(ge, mid, lo), jnp.where(ge, hi, mid - 1)

    def sbody(carry):
        lo, hi = sstep(*carry)
        return sstep(lo, hi)

    t_key, _ = lax.while_loop(scond, sbody, (g51, hi0))

    z = jnp.maximum(_from_key(hi0), pos_max)
    keys = keys_ref[...]
    gt = keys > t_key
    cnt_gt = jnp.sum(gt.astype(jnp.float32), axis=1, keepdims=True)
    se_gt = jnp.sum(jnp.where(gt, jnp.exp(_from_key(keys) - z), 0.0),
                    axis=1, keepdims=True)
    kneg_f = kneg.astype(jnp.float32)
    se_neg = se_gt + (kneg_f - cnt_gt) * jnp.exp(_from_key(t_key) - z)

    se_total = pos_se * jnp.exp(pos_max - z) + se_neg
    lse = z + jnp.log(se_total)
    n_pos_f = n_pos.astype(jnp.float32)
    k_pos = jnp.minimum(n_pos_f, jnp.float32(NEGK))
    loss_rows = (k_pos / n_pos_f) * lse - sum_pos / n_pos_f
    out_ref[...] = jnp.sum(loss_rows).reshape(1, 1, 1)


def _build_loss_call(b, m, d, rb, interpret=False):
    grid = (b // rb,)
    return pl.pallas_call(
        _loss_block,
        grid=grid,
        in_specs=[
            pl.BlockSpec((rb, d), lambda r: (r, 0)),
            pl.BlockSpec((m, d), lambda r: (0, 0)),
            pl.BlockSpec((1, m), lambda r: (0, 0)),
            pl.BlockSpec((rb, 1), lambda r: (r, 0)),
        ],
        out_specs=pl.BlockSpec((1, 1, 1), lambda r: (r, 0, 0)),
        out_shape=jax.ShapeDtypeStruct((b // rb, 1, 1), jnp.float32),
        scratch_shapes=[pltpu.VMEM((rb, m), jnp.int32)],
        interpret=interpret,
    )


def _build_label_gather(b):
    info = plsc.get_sparse_core_info()
    nw = info.num_cores * info.num_subcores
    b_per_w = b // nw

    @functools.partial(
        pl.kernel,
        mesh=plsc.VectorSubcoreMesh(core_axis_name="c", subcore_axis_name="s"),
        out_type=jax.ShapeDtypeStruct((b,), jnp.int32),
        scratch_types=[
            pltpu.VMEM((b_per_w,), jnp.int32),
            pltpu.VMEM((b_per_w,), jnp.int32),
            pltpu.VMEM((b_per_w,), jnp.int32),
            pltpu.SemaphoreType.DMA,
        ],
    )
    def gather_kernel(labels_hbm, img_hbm, plab_hbm, out_hbm,
                      idx_v, pidx_v, lab_v, sem):
        wid = lax.axis_index("s") * info.num_cores + lax.axis_index("c")
        base = wid * b_per_w
        pltpu.sync_copy(labels_hbm.at[pl.ds(base, b_per_w)], idx_v)
        pltpu.async_copy(img_hbm.at[idx_v], pidx_v, sem).wait()
        pltpu.async_copy(plab_hbm.at[pidx_v], lab_v, sem).wait()
        pltpu.sync_copy(lab_v, out_hbm.at[pl.ds(base, b_per_w)])

    return gather_kernel


def kernel(features, index_labels, proxy_memory, img_proxy_index, all_proxy_label):
    b, d = features.shape
    m = proxy_memory.shape[0]
    rb = min(128, b)
    batch_pseudo_label = _build_label_gather(b)(
        index_labels, img_proxy_index, all_proxy_label)
    call = _build_loss_call(b, m, d, rb)
    partial = call(
        features, proxy_memory, all_proxy_label.reshape(1, m),
        batch_pseudo_label.reshape(b, 1))
    return jnp.sum(partial) / b


# speedup vs baseline: 1.1956x; 1.1956x over previous
import functools

import jax
import jax.numpy as jnp
from jax import lax
from jax.experimental import pallas as pl
from jax.experimental.pallas import tpu as pltpu
from jax.experimental.pallas import tpu_sc as plsc

NEGK = 50
TEMP = 0.05
NEG_FILL = -3.0e38
IMIN = -2147483648
IMAX = 2147483647


def _to_key(s):
    bits = lax.bitcast_convert_type(s, jnp.int32)
    return jnp.where(bits < 0, bits ^ IMAX, bits)


def _from_key(k):
    bits = jnp.where(k < 0, k ^ IMAX, k)
    return lax.bitcast_convert_type(bits, jnp.float32)


def _mid(lo, hi):
    return (lo >> 1) + (hi >> 1) + (lo & hi & 1) + ((lo ^ hi) & 1)


def _loss_block(feat_ref, pm_ref, labels_ref, blab_ref, out_ref, keys_ref):
    rb, d = feat_ref.shape
    m = pm_ref.shape[0]
    cb = min(512, m)
    n_chunks = m // cb

    f = feat_ref[...] * (1.0 / TEMP)
    blab = blab_ref[...]

    def chunk(c, carry):
        n_pos, sum_pos, pos_max, pos_se, g = carry
        pm_c = pm_ref[pl.ds(c * cb, cb), :]
        s = lax.dot_general(f, pm_c, (((1,), (1,)), ((), ())),
                            preferred_element_type=jnp.float32)
        mask = labels_ref[:, pl.ds(c * cb, cb)] == blab
        key = jnp.where(mask, IMIN, _to_key(s))
        keys_ref[:, pl.ds(c * cb, cb)] = key
        g = jnp.maximum(g, key)
        n_pos = n_pos + jnp.sum(mask.astype(jnp.int32), axis=1, keepdims=True)
        sum_pos = sum_pos + jnp.sum(jnp.where(mask, s, 0.0), axis=1, keepdims=True)
        m_new = jnp.maximum(
            pos_max, jnp.max(jnp.where(mask, s, NEG_FILL), axis=1, keepdims=True))
        pos_se = pos_se * jnp.exp(pos_max - m_new) + jnp.sum(
            jnp.where(mask, jnp.exp(s - m_new), 0.0), axis=1, keepdims=True)
        return n_pos, sum_pos, m_new, pos_se, g

    zero = jnp.zeros((rb, 1), jnp.float32)
    n_pos, sum_pos, pos_max, pos_se, g = lax.fori_loop(
        0, n_chunks, chunk,
        (jnp.zeros((rb, 1), jnp.int32), zero, jnp.full((rb, 1), NEG_FILL), zero,
         jnp.full((rb, cb), IMIN, jnp.int32)))

    hi0 = jnp.max(g, axis=1, keepdims=True)
    kneg = jnp.maximum(NEGK - jnp.minimum(n_pos, NEGK), 0)

    def gsearch(_, carry):
        lo, hi = carry
        mid = _mid(lo, hi)
        cnt = jnp.sum((g >= mid).astype(jnp.int32), axis=1, keepdims=True)
        ge = cnt >= (NEGK + 1)
        return jnp.where(ge, mid, lo), jnp.where(ge, hi, mid - 1)

    g51, _ = lax.fori_loop(
        0, 32, gsearch, (jnp.min(g, axis=1, keepdims=True), hi0))

    def scond(carry):
        lo, hi = carry
        return jnp.any(lo < hi)

    def sbody(carry):
        lo, hi = carry
        mid = _mid(lo, hi)
        cnt = jnp.sum((keys_ref[...] >= mid).astype(jnp.int32),
                      axis=1, keepdims=True)
        ge = cnt >= kneg
        return jnp.where(ge, mid, lo), jnp.where(ge, hi, mid - 1)

    t_key, _ = lax.while_loop(scond, sbody, (g51, hi0))

    z = jnp.maximum(_from_key(hi0), pos_max)
    keys = keys_ref[...]
    gt = keys > t_key
    cnt_gt = jnp.sum(gt.astype(jnp.float32), axis=1, keepdims=True)
    se_gt = jnp.sum(jnp.where(gt, jnp.exp(_from_key(keys) - z), 0.0),
                    axis=1, keepdims=True)
    kneg_f = kneg.astype(jnp.float32)
    se_neg = se_gt + (kneg_f - cnt_gt) * jnp.exp(_from_key(t_key) - z)

    se_total = pos_se * jnp.exp(pos_max - z) + se_neg
    lse = z + jnp.log(se_total)
    n_pos_f = n_pos.astype(jnp.float32)
    k_pos = jnp.minimum(n_pos_f, jnp.float32(NEGK))
    loss_rows = (k_pos / n_pos_f) * lse - sum_pos / n_pos_f
    out_ref[...] = jnp.sum(loss_rows).reshape(1, 1, 1)


def _build_loss_call(b, m, d, rb, interpret=False):
    grid = (b // rb,)
    return pl.pallas_call(
        _loss_block,
        grid=grid,
        in_specs=[
            pl.BlockSpec((rb, d), lambda r: (r, 0)),
            pl.BlockSpec((m, d), lambda r: (0, 0)),
            pl.BlockSpec((1, m), lambda r: (0, 0)),
            pl.BlockSpec((rb, 1), lambda r: (r, 0)),
        ],
        out_specs=pl.BlockSpec((1, 1, 1), lambda r: (r, 0, 0)),
        out_shape=jax.ShapeDtypeStruct((b // rb, 1, 1), jnp.float32),
        scratch_shapes=[pltpu.VMEM((rb, m), jnp.int32)],
        interpret=interpret,
    )


def _build_label_gather(b):
    info = plsc.get_sparse_core_info()
    nw = info.num_cores * info.num_subcores
    b_per_w = b // nw

    @functools.partial(
        pl.kernel,
        mesh=plsc.VectorSubcoreMesh(core_axis_name="c", subcore_axis_name="s"),
        out_type=jax.ShapeDtypeStruct((b,), jnp.int32),
        scratch_types=[
            pltpu.VMEM((b_per_w,), jnp.int32),
            pltpu.VMEM((b_per_w,), jnp.int32),
            pltpu.VMEM((b_per_w,), jnp.int32),
            pltpu.SemaphoreType.DMA,
        ],
    )
    def gather_kernel(labels_hbm, img_hbm, plab_hbm, out_hbm,
                      idx_v, pidx_v, lab_v, sem):
        wid = lax.axis_index("s") * info.num_cores + lax.axis_index("c")
        base = wid * b_per_w
        pltpu.sync_copy(labels_hbm.at[pl.ds(base, b_per_w)], idx_v)
        pltpu.async_copy(img_hbm.at[idx_v], pidx_v, sem).wait()
        pltpu.async_copy(plab_hbm.at[pidx_v], lab_v, sem).wait()
        pltpu.sync_copy(lab_v, out_hbm.at[pl.ds(base, b_per_w)])

    return gather_kernel


def kernel(features, index_labels, proxy_memory, img_proxy_index, all_proxy_label):
    b, d = features.shape
    m = proxy_memory.shape[0]
    rb = min(256, b)
    batch_pseudo_label = _build_label_gather(b)(
        index_labels, img_proxy_index, all_proxy_label)
    call = _build_loss_call(b, m, d, rb)
    partial = call(
        features, proxy_memory, all_proxy_label.reshape(1, m),
        batch_pseudo_label.reshape(b, 1))
    return jnp.sum(partial) / b


# speedup vs baseline: 1.2733x; 1.0650x over previous
import functools

import jax
import jax.numpy as jnp
from jax import lax
from jax.experimental import pallas as pl
from jax.experimental.pallas import tpu as pltpu
from jax.experimental.pallas import tpu_sc as plsc

NEGK = 50
TEMP = 0.05
NEG_FILL = -3.0e38
IMIN = -2147483648
IMAX = 2147483647


def _to_key(s):
    bits = lax.bitcast_convert_type(s, jnp.int32)
    return jnp.where(bits < 0, bits ^ IMAX, bits)


def _from_key(k):
    bits = jnp.where(k < 0, k ^ IMAX, k)
    return lax.bitcast_convert_type(bits, jnp.float32)


def _mid(lo, hi):
    return (lo >> 1) + (hi >> 1) + (lo & hi & 1) + ((lo ^ hi) & 1)


def _loss_block(feat_ref, pm_ref, labels_ref, blab_ref, out_ref, keys_ref):
    rb, d = feat_ref.shape
    m = pm_ref.shape[0]
    cb = min(512, m)
    n_chunks = m // cb

    f = feat_ref[...] * (1.0 / TEMP)
    blab = blab_ref[...]

    def chunk(c, carry):
        n_pos, sum_pos, pos_max, pos_se, g = carry
        pm_c = pm_ref[pl.ds(c * cb, cb), :]
        s = lax.dot_general(f, pm_c, (((1,), (1,)), ((), ())),
                            preferred_element_type=jnp.float32)
        mask = labels_ref[:, pl.ds(c * cb, cb)] == blab
        key = jnp.where(mask, IMIN, _to_key(s))
        keys_ref[:, pl.ds(c * cb, cb)] = key
        g = jnp.maximum(g, key)
        n_pos = n_pos + jnp.sum(mask.astype(jnp.int32), axis=1, keepdims=True)
        sum_pos = sum_pos + jnp.sum(jnp.where(mask, s, 0.0), axis=1, keepdims=True)
        m_new = jnp.maximum(
            pos_max, jnp.max(jnp.where(mask, s, NEG_FILL), axis=1, keepdims=True))
        pos_se = pos_se * jnp.exp(pos_max - m_new) + jnp.sum(
            jnp.where(mask, jnp.exp(s - m_new), 0.0), axis=1, keepdims=True)
        return n_pos, sum_pos, m_new, pos_se, g

    zero = jnp.zeros((rb, 1), jnp.float32)
    n_pos, sum_pos, pos_max, pos_se, g = lax.fori_loop(
        0, n_chunks, chunk,
        (jnp.zeros((rb, 1), jnp.int32), zero, jnp.full((rb, 1), NEG_FILL), zero,
         jnp.full((rb, cb), IMIN, jnp.int32)))

    hi0 = jnp.max(g, axis=1, keepdims=True)
    kneg = jnp.maximum(NEGK - jnp.minimum(n_pos, NEGK), 0)

    def gsearch(_, carry):
        lo, hi = carry
        mid = _mid(lo, hi)
        cnt = jnp.sum((g >= mid).astype(jnp.int32), axis=1, keepdims=True)
        ge = cnt >= (NEGK + 1)
        return jnp.where(ge, mid, lo), jnp.where(ge, hi, mid - 1)

    g51, _ = lax.fori_loop(
        0, 32, gsearch, (jnp.min(g, axis=1, keepdims=True), hi0))

    def scond(carry):
        lo, hi = carry
        return jnp.any(lo < hi)

    def sbody(carry):
        lo, hi = carry
        mid = _mid(lo, hi)
        cnt = jnp.sum((keys_ref[...] >= mid).astype(jnp.int32),
                      axis=1, keepdims=True)
        ge = cnt >= kneg
        return jnp.where(ge, mid, lo), jnp.where(ge, hi, mid - 1)

    t_key, _ = lax.while_loop(scond, sbody, (g51, hi0))

    z = jnp.maximum(_from_key(hi0), pos_max)
    keys = keys_ref[...]
    gt = keys > t_key
    cnt_gt = jnp.sum(gt.astype(jnp.float32), axis=1, keepdims=True)
    se_gt = jnp.sum(jnp.where(gt, jnp.exp(_from_key(keys) - z), 0.0),
                    axis=1, keepdims=True)
    kneg_f = kneg.astype(jnp.float32)
    se_neg = se_gt + (kneg_f - cnt_gt) * jnp.exp(_from_key(t_key) - z)

    se_total = pos_se * jnp.exp(pos_max - z) + se_neg
    lse = z + jnp.log(se_total)
    n_pos_f = n_pos.astype(jnp.float32)
    k_pos = jnp.minimum(n_pos_f, jnp.float32(NEGK))
    loss_rows = (k_pos / n_pos_f) * lse - sum_pos / n_pos_f
    out_ref[...] = jnp.sum(loss_rows).reshape(1, 1, 1)


def _build_loss_call(b, m, d, rb, interpret=False):
    grid = (b // rb,)
    return pl.pallas_call(
        _loss_block,
        grid=grid,
        in_specs=[
            pl.BlockSpec((rb, d), lambda r: (r, 0)),
            pl.BlockSpec((m, d), lambda r: (0, 0)),
            pl.BlockSpec((1, m), lambda r: (0, 0)),
            pl.BlockSpec((rb, 1), lambda r: (r, 0)),
        ],
        out_specs=pl.BlockSpec((1, 1, 1), lambda r: (r, 0, 0)),
        out_shape=jax.ShapeDtypeStruct((b // rb, 1, 1), jnp.float32),
        scratch_shapes=[pltpu.VMEM((rb, m), jnp.int32)],
        interpret=interpret,
    )


def _build_label_gather(b):
    info = plsc.get_sparse_core_info()
    nw = info.num_cores * info.num_subcores
    b_per_w = b // nw

    @functools.partial(
        pl.kernel,
        mesh=plsc.VectorSubcoreMesh(core_axis_name="c", subcore_axis_name="s"),
        out_type=jax.ShapeDtypeStruct((b,), jnp.int32),
        scratch_types=[
            pltpu.VMEM((b_per_w,), jnp.int32),
            pltpu.VMEM((b_per_w,), jnp.int32),
            pltpu.VMEM((b_per_w,), jnp.int32),
            pltpu.SemaphoreType.DMA,
        ],
    )
    def gather_kernel(labels_hbm, img_hbm, plab_hbm, out_hbm,
                      idx_v, pidx_v, lab_v, sem):
        wid = lax.axis_index("s") * info.num_cores + lax.axis_index("c")
        base = wid * b_per_w
        pltpu.sync_copy(labels_hbm.at[pl.ds(base, b_per_w)], idx_v)
        pltpu.async_copy(img_hbm.at[idx_v], pidx_v, sem).wait()
        pltpu.async_copy(plab_hbm.at[pidx_v], lab_v, sem).wait()
        pltpu.sync_copy(lab_v, out_hbm.at[pl.ds(base, b_per_w)])

    return gather_kernel


def kernel(features, index_labels, proxy_memory, img_proxy_index, all_proxy_label):
    b, d = features.shape
    m = proxy_memory.shape[0]
    rb = min(512, b)
    batch_pseudo_label = _build_label_gather(b)(
        index_labels, img_proxy_index, all_proxy_label)
    call = _build_loss_call(b, m, d, rb)
    partial = call(
        features, proxy_memory, all_proxy_label.reshape(1, m),
        batch_pseudo_label.reshape(b, 1))
    return jnp.sum(partial) / b
